# baseline (speedup 1.0000x reference)
import functools

import jax
import jax.numpy as jnp
from jax import lax
from jax.experimental import pallas as pl
from jax.experimental.pallas import tpu as pltpu
from jax.experimental.pallas import tpu_sc as plsc

VOCAB = 4
SEQ = 10
D = 64
BATCH = 16384

_info = plsc.get_sparse_core_info()
_NC = _info.num_cores
_NS = _info.num_subcores
_NW = _NC * _NS
_L = 16

BW = BATCH // _NW
D8 = 8


def _lut_body(tok_ref, pos_ref, gamma_ref, beta_ref, lut_ref):
    tok = tok_ref[:, :]
    pos = pos_ref[:, :]
    e = tok[:, None, :] + pos[None, :, :]
    mean = jnp.mean(e, axis=-1, keepdims=True)
    c = e - mean
    var = jnp.mean(c * c, axis=-1, keepdims=True)
    normed = c * lax.rsqrt(var + 1e-5)
    lut_ref[:, :, :] = (normed * gamma_ref[0][None, None, :]
                        + beta_ref[0][None, None, :])


_lut_call = pl.pallas_call(
    _lut_body,
    out_shape=jax.ShapeDtypeStruct((VOCAB, SEQ, D), jnp.float32),
)


_sc_mesh = plsc.VectorSubcoreMesh(core_axis_name="c", subcore_axis_name="s")


@functools.partial(
    pl.kernel,
    mesh=_sc_mesh,
    out_type=jax.ShapeDtypeStruct((SEQ, D, BATCH), jnp.float32),
    scratch_types=[
        pltpu.VMEM((SEQ * D * VOCAB * _L // 128, 128), jnp.float32),
        pltpu.VMEM((SEQ, BW), jnp.int32),
        pltpu.VMEM((D, BW), jnp.float32),
        pltpu.VMEM((D, BW), jnp.float32),
        pltpu.SemaphoreType.DMA,
        pltpu.SemaphoreType.DMA,
        pltpu.SemaphoreType.DMA,
    ],
)
def _sc_select(rep_hbm, xt_hbm, out_hbm, rep_v, x_v, slab0, slab1,
               ssem, osem0, osem1):
    wid = lax.axis_index("s") * _NC + lax.axis_index("c")
    b0 = wid * BW

    stage = [pltpu.make_async_copy(rep_hbm, rep_v, ssem)]
    stage += [
        pltpu.make_async_copy(xt_hbm.at[pl.ds(s * BATCH + b0, BW)],
                              x_v.at[s], ssem)
        for s in range(SEQ)
    ]
    for cp in stage:
        cp.start()
    stage[0].wait()
    stage[1].wait()

    slabs = (slab0, slab1)
    osem = (osem0, osem1)
    ocp = [None, None]
    for s in range(SEQ):
        if s > 0:
            stage[1 + s].wait()
        sb = s % 2
        if ocp[sb] is not None:
            ocp[sb].wait()
        slab = slabs[sb]
        for dblk in range(D // D8):
            A = []
            for j in range(D8):
                flat = ((s * D + dblk * D8 + j) * VOCAB) * _L
                A.append([rep_v[(flat + v * _L) // 128,
                                pl.ds((flat + v * _L) % 128, _L)]
                          for v in range(VOCAB)])

            def body(b16, carry, s=s, dblk=dblk, A=A, slab=slab):
                off = b16 * _L
                c = x_v[s, pl.ds(off, _L)]
                m1 = c == 1
                m2 = c == 2
                m3 = c == 3
                for j in range(D8):
                    r = jnp.where(m1, A[j][1], A[j][0])
                    r = jnp.where(m2, A[j][2], r)
                    r = jnp.where(m3, A[j][3], r)
                    slab[dblk * D8 + j, pl.ds(off, _L)] = r
                return carry

            lax.fori_loop(0, BW // _L, body, 0)
        ocp[sb] = pltpu.make_async_copy(
            slab, out_hbm.at[s, :, pl.ds(b0, BW)], osem[sb])
        ocp[sb].start()
    ocp[0].wait()
    ocp[1].wait()


def kernel(x, tok_table, pos_table, gamma, beta):
    lut = _lut_call(tok_table, pos_table,
                    gamma.reshape(1, D), beta.reshape(1, D))
    rep = jnp.broadcast_to(
        lut.transpose(1, 2, 0)[:, :, :, None], (SEQ, D, VOCAB, _L)
    ).reshape(SEQ * D * VOCAB * _L // 128, 128)
    xt = x.T.reshape(SEQ * BATCH)
    out = _sc_select(rep, xt)
    return jnp.transpose(out, (2, 0, 1))


# speedup vs baseline: 1.0133x; 1.0133x over previous
import functools

import jax
import jax.numpy as jnp
from jax import lax
from jax.experimental import pallas as pl
from jax.experimental.pallas import tpu as pltpu
from jax.experimental.pallas import tpu_sc as plsc

VOCAB = 4
SEQ = 10
D = 64
BATCH = 16384

_info = plsc.get_sparse_core_info()
_NC = _info.num_cores
_NS = _info.num_subcores
_NW = _NC * _NS
_L = 16

BW = BATCH // _NW
D8 = 8


def _lut_body(tok_ref, pos_ref, gamma_ref, beta_ref, lut_ref):
    tok = tok_ref[:, :]
    pos = pos_ref[:, :]
    e = tok[:, None, :] + pos[None, :, :]
    mean = jnp.mean(e, axis=-1, keepdims=True)
    c = e - mean
    var = jnp.mean(c * c, axis=-1, keepdims=True)
    normed = c * lax.rsqrt(var + 1e-5)
    lut_ref[:, :, :] = (normed * gamma_ref[0][None, None, :]
                        + beta_ref[0][None, None, :])


_lut_call = pl.pallas_call(
    _lut_body,
    out_shape=jax.ShapeDtypeStruct((VOCAB, SEQ, D), jnp.float32),
)


_sc_mesh = plsc.VectorSubcoreMesh(core_axis_name="c", subcore_axis_name="s")


@functools.partial(
    pl.kernel,
    mesh=_sc_mesh,
    out_type=jax.ShapeDtypeStruct((SEQ, D, BATCH), jnp.float32),
    scratch_types=[
        pltpu.VMEM((SEQ * D * VOCAB * _L // 128, 128), jnp.float32),
        pltpu.VMEM((SEQ, BW), jnp.int32),
        pltpu.VMEM((D, BW), jnp.float32),
        pltpu.VMEM((D, BW), jnp.float32),
        pltpu.SemaphoreType.DMA,
        pltpu.SemaphoreType.DMA,
        pltpu.SemaphoreType.DMA,
    ],
)
def _sc_select(rep_hbm, xt_hbm, out_hbm, rep_v, x_v, slab0, slab1,
               ssem, osem0, osem1):
    wid = lax.axis_index("s") * _NC + lax.axis_index("c")
    b0 = wid * BW

    stage = [pltpu.make_async_copy(rep_hbm, rep_v, ssem)]
    stage += [
        pltpu.make_async_copy(xt_hbm.at[pl.ds(s * BATCH + b0, BW)],
                              x_v.at[s], ssem)
        for s in range(SEQ)
    ]
    for cp in stage:
        cp.start()
    stage[0].wait()
    stage[1].wait()

    slabs = (slab0, slab1)
    osem = (osem0, osem1)
    ocp = [None, None]
    for s in range(SEQ):
        if s > 0:
            stage[1 + s].wait()
        sb = s % 2
        if ocp[sb] is not None:
            ocp[sb].wait()
        slab = slabs[sb]
        for dblk in range(D // D8):
            A = [[rep_v[v * (SEQ * D // 8) + s * (D // 8) + dblk,
                        pl.ds(j * _L, _L)]
                  for v in range(VOCAB)] for j in range(D8)]

            def body(b16, carry, s=s, dblk=dblk, A=A, slab=slab):
                off = b16 * _L
                c = x_v[s, pl.ds(off, _L)]
                m1 = c == 1
                m2 = c == 2
                m3 = c == 3
                for j in range(D8):
                    r = jnp.where(m1, A[j][1], A[j][0])
                    r = jnp.where(m2, A[j][2], r)
                    r = jnp.where(m3, A[j][3], r)
                    slab[dblk * D8 + j, pl.ds(off, _L)] = r
                return carry

            lax.fori_loop(0, BW // _L, body, 0)
        ocp[sb] = pltpu.make_async_copy(
            slab, out_hbm.at[s, :, pl.ds(b0, BW)], osem[sb])
        ocp[sb].start()
    ocp[0].wait()
    ocp[1].wait()


def kernel(x, tok_table, pos_table, gamma, beta):
    lut = _lut_call(tok_table, pos_table,
                    gamma.reshape(1, D), beta.reshape(1, D))
    rep = jnp.broadcast_to(
        lut[:, :, :, None], (VOCAB, SEQ, D, _L)
    ).reshape(SEQ * D * VOCAB * _L // 128, 128)
    xt = x.T.reshape(SEQ * BATCH)
    out = _sc_select(rep, xt)
    return jnp.transpose(out, (2, 0, 1))


# speedup vs baseline: 1.0388x; 1.0251x over previous
import functools

import jax
import jax.numpy as jnp
from jax import lax
from jax.experimental import pallas as pl
from jax.experimental.pallas import tpu as pltpu
from jax.experimental.pallas import tpu_sc as plsc

VOCAB = 4
SEQ = 10
D = 64
BATCH = 16384

_info = plsc.get_sparse_core_info()
_NC = _info.num_cores
_NS = _info.num_subcores
_NW = _NC * _NS
_L = 16

BW = BATCH // _NW
D8 = 8


def _lut_body(tok_ref, pos_ref, gamma_ref, beta_ref, lut_ref):
    tok = tok_ref[:, :]
    pos = pos_ref[:, :]
    e = tok[:, None, :] + pos[None, :, :]
    mean = jnp.mean(e, axis=-1, keepdims=True)
    c = e - mean
    var = jnp.mean(c * c, axis=-1, keepdims=True)
    normed = c * lax.rsqrt(var + 1e-5)
    lut_ref[:, :, :] = (normed * gamma_ref[0][None, None, :]
                        + beta_ref[0][None, None, :])


_lut_call = pl.pallas_call(
    _lut_body,
    out_shape=jax.ShapeDtypeStruct((VOCAB, SEQ, D), jnp.float32),
)


_sc_mesh = plsc.VectorSubcoreMesh(core_axis_name="c", subcore_axis_name="s")


@functools.partial(
    pl.kernel,
    mesh=_sc_mesh,
    out_type=jax.ShapeDtypeStruct((SEQ, D, BATCH), jnp.float32),
    scratch_types=[
        pltpu.VMEM((SEQ * D * VOCAB * _L // 128, 128), jnp.float32),
        pltpu.VMEM((SEQ, BW), jnp.int32),
        pltpu.VMEM((D, BW), jnp.float32),
        pltpu.VMEM((D, BW), jnp.float32),
        pltpu.SemaphoreType.DMA,
        pltpu.SemaphoreType.DMA,
        pltpu.SemaphoreType.DMA,
    ],
)
def _sc_select(rep_hbm, xt_hbm, out_hbm, rep_v, x_v, slab0, slab1,
               ssem, osem0, osem1):
    wid = lax.axis_index("s") * _NC + lax.axis_index("c")
    b0 = wid * BW

    stage = [pltpu.make_async_copy(rep_hbm, rep_v, ssem)]
    stage += [
        pltpu.make_async_copy(xt_hbm.at[s, pl.ds(b0, BW)], x_v.at[s], ssem)
        for s in range(SEQ)
    ]
    for cp in stage:
        cp.start()
    stage[0].wait()
    stage[1].wait()

    slabs = (slab0, slab1)
    osem = (osem0, osem1)
    ocp = [None, None]
    for s in range(SEQ):
        if s > 0:
            stage[1 + s].wait()
        sb = s % 2
        if ocp[sb] is not None:
            ocp[sb].wait()
        slab = slabs[sb]
        for dblk in range(D // D8):
            A = [[rep_v[v * (SEQ * D // 8) + s * (D // 8) + dblk,
                        pl.ds(j * _L, _L)]
                  for v in range(VOCAB)] for j in range(D8)]

            def body(b16, carry, s=s, dblk=dblk, A=A, slab=slab):
                off = b16 * _L
                c = x_v[s, pl.ds(off, _L)]
                m1 = c == 1
                m2 = c == 2
                m3 = c == 3
                for j in range(D8):
                    r = jnp.where(m1, A[j][1], A[j][0])
                    r = jnp.where(m2, A[j][2], r)
                    r = jnp.where(m3, A[j][3], r)
                    slab[dblk * D8 + j, pl.ds(off, _L)] = r
                return carry

            lax.fori_loop(0, BW // _L, body, 0)
        ocp[sb] = pltpu.make_async_copy(
            slab, out_hbm.at[s, :, pl.ds(b0, BW)], osem[sb])
        ocp[sb].start()
    ocp[0].wait()
    ocp[1].wait()


def kernel(x, tok_table, pos_table, gamma, beta):
    lut = _lut_call(tok_table, pos_table,
                    gamma.reshape(1, D), beta.reshape(1, D))
    rep = jnp.broadcast_to(
        lut[:, :, :, None], (VOCAB, SEQ, D, _L)
    ).reshape(SEQ * D * VOCAB * _L // 128, 128)
    out = _sc_select(rep, x.T)
    return jnp.transpose(out, (2, 0, 1))
